# Initial kernel scaffold; baseline (speedup 1.0000x reference)
import jax
import jax.numpy as jnp
from jax.experimental import pallas as pl


def kernel(x, edge_index, batch_size, W1, b1, W2, b2, W3, b3, W4, b4, W5, b5, lin_w, lin_b):
    raise NotImplementedError("write your pallas kernel here")


# speedup vs baseline: 10.3769x; 10.3769x over previous
import functools

import jax
import jax.numpy as jnp
from jax import lax
from jax.experimental import pallas as pl
from jax.experimental.pallas import tpu as pltpu
from jax.experimental.pallas import tpu_sc as plsc

N = 50000
NP = 51200
E = 800000
EP = 819200
EROWS = EP // 128
ROWS_T = NP // 16
RB = 4
CH = 32

_f32 = jnp.float32
_i32 = jnp.int32

_BN = 512
_GRID = NP // _BN

@functools.lru_cache(maxsize=None)
def _sc_pass(num_groups, do_gather):
    mesh = plsc.VectorSubcoreMesh(core_axis_name="c", subcore_axis_name="s")
    out_type = jax.ShapeDtypeStruct((2 * NP, CH), _f32)
    scratch = [
        pltpu.VMEM_SHARED((NP, CH), _f32),
        pltpu.VMEM((RB, 128), _i32),
        pltpu.VMEM((RB, 128), _i32),
        pltpu.VMEM((RB * 128, CH), _f32),
        pltpu.SemaphoreType.DMA,
        pltpu.SemaphoreType.DMA,
    ]

    @functools.partial(
        pl.kernel, out_type=out_type, mesh=mesh, scratch_types=scratch,
        compiler_params=pltpu.CompilerParams(use_tc_tiling_on_sc=False))
    def k(z_hbm, src_hbm, dst_hbm, zeros_hbm, out_hbm,
          acc, src_v, dst_v, gbuf, sem_g, sem_s):
        c = lax.axis_index("c")
        s = lax.axis_index("s")
        pltpu.sync_copy(zeros_hbm, gbuf)
        for kk in range(ROWS_T // 400):
            pltpu.sync_copy(gbuf.at[pl.ds(0, 400)],
                            acc.at[pl.ds(s * ROWS_T + kk * 400, 400)])
        plsc.subcore_barrier()
        if not do_gather:
            pltpu.sync_copy(z_hbm, gbuf)
        if num_groups == 1:
            w = s * 2 + c
            row0 = w * (EROWS // 32)
            drow0 = row0
            iters = EROWS // 32 // RB
        else:
            drow0 = s * (EROWS // 16)
            row0 = c * EROWS + drow0
            iters = EROWS // 16 // RB
        obase = c * NP

        def body(i, carry):
            r0 = row0 + i * RB
            rd = drow0 + i * RB
            pltpu.sync_copy(src_hbm.at[pl.ds(r0, RB)], src_v)
            pltpu.sync_copy(dst_hbm.at[pl.ds(rd, RB)], dst_v)
            if do_gather:
                hs = [pltpu.async_copy(z_hbm.at[src_v.at[j]],
                                       gbuf.at[pl.ds(j * 128, 128)],
                                       sem_g)
                      for j in range(RB)]
                for h in hs:
                    h.wait()
            for j in range(RB):
                pltpu.sync_copy(gbuf.at[pl.ds(j * 128, 128)],
                                acc.at[dst_v.at[j]], add=True)
            return carry

        lax.fori_loop(0, iters, body, 0)
        plsc.subcore_barrier()
        for kk in range(ROWS_T // 400):
            r = s * ROWS_T + kk * 400
            pltpu.sync_copy(acc.at[pl.ds(r, 400)], gbuf.at[pl.ds(0, 400)])
            pltpu.sync_copy(gbuf.at[pl.ds(0, 400)],
                            out_hbm.at[pl.ds(obase + r, 400)])

    return k


def _rows(w):
    return pl.BlockSpec((_BN, w), lambda i: (i, 0))


def _full(shape):
    return pl.BlockSpec(shape, lambda i: (0,) * len(shape))


def _tc_a(PA, x_pad):
    def body(p0, p1, xp, dis_o, xs_o):
        i = pl.program_id(0)
        indeg = p0[:, 0:1] + p1[:, 0:1]
        rows = i * _BN + lax.broadcasted_iota(_i32, (_BN, 1), 0)
        dis = jnp.where(rows < N, lax.rsqrt(indeg + 1.0), 0.0)
        dis_o[...] = dis
        xs_o[...] = xp[...] * dis

    return pl.pallas_call(
        body, grid=(_GRID,),
        in_specs=[pl.BlockSpec((_BN, CH), lambda i: (i, 0)),
                  pl.BlockSpec((_BN, CH), lambda i: (i + _GRID, 0)),
                  _rows(CH)],
        out_specs=[pl.BlockSpec((_BN, 1), lambda i: (i, 0)), _rows(CH)],
        out_shape=[jax.ShapeDtypeStruct((NP, 1), _f32),
                   jax.ShapeDtypeStruct((NP, CH), _f32)],
    )(PA, PA, x_pad)


def _tc_b(PB, xs, dis, W1p, b1, W2):
    def body(p0, p1, xs_r, dis_r, w1_r, b1_r, w2_r, out_r):
        d = dis_r[...]
        agg = d * (p0[...] + p1[...] + xs_r[...])
        h1 = jnp.maximum(
            jnp.dot(agg, w1_r[...], preferred_element_type=_f32) + b1_r[...],
            0.0)
        z2 = jnp.dot(h1, w2_r[...], preferred_element_type=_f32) * d
        for g in range(4):
            out_r[g] = z2[:, g * CH:(g + 1) * CH]

    return pl.pallas_call(
        body, grid=(_GRID,),
        in_specs=[pl.BlockSpec((_BN, CH), lambda i: (i, 0)),
                  pl.BlockSpec((_BN, CH), lambda i: (i + _GRID, 0)),
                  _rows(CH), _rows(1),
                  _full((CH, 128)), _full((1, 128)), _full((128, 128))],
        out_specs=pl.BlockSpec((4, _BN, CH), lambda i: (0, i, 0)),
        out_shape=jax.ShapeDtypeStruct((4, NP, CH), _f32),
    )(PB, PB, xs, dis, W1p, b1, W2)


def _tc_c(S2a, S2b, z2g, dis, b2, W3):
    def body(sa_r, sb_r, z2_r, dis_r, b2_r, w3_r, out_r):
        d = dis_r[...]
        s2 = jnp.concatenate([sa_r[0], sa_r[1], sb_r[0], sb_r[1]], axis=1)
        z2 = jnp.concatenate([z2_r[g] for g in range(4)], axis=1)
        h2 = jnp.maximum(d * (s2 + z2) + b2_r[...], 0.0)
        out_r[...] = jnp.dot(h2, w3_r[...], preferred_element_type=_f32) * d

    return pl.pallas_call(
        body, grid=(_GRID,),
        in_specs=[pl.BlockSpec((2, _BN, CH), lambda i: (0, i, 0)),
                  pl.BlockSpec((2, _BN, CH), lambda i: (0, i, 0)),
                  pl.BlockSpec((4, _BN, CH), lambda i: (0, i, 0)),
                  _rows(1), _full((1, 128)), _full((128, CH))],
        out_specs=_rows(CH),
        out_shape=jax.ShapeDtypeStruct((NP, CH), _f32),
    )(S2a, S2b, z2g, dis, b2, W3)


def _tc_mid(P, zp, dis, b, Wn):
    def body(p_r, z_r, dis_r, b_r, w_r, out_r):
        d = dis_r[...]
        h = jnp.maximum(d * (p_r[0] + p_r[1] + z_r[...]) + b_r[...], 0.0)
        out_r[...] = jnp.dot(h, w_r[...], preferred_element_type=_f32) * d

    return pl.pallas_call(
        body, grid=(_GRID,),
        in_specs=[pl.BlockSpec((2, _BN, CH), lambda i: (0, i, 0)),
                  _rows(CH), _rows(1), _full((1, CH)), _full((CH, CH))],
        out_specs=_rows(CH),
        out_shape=jax.ShapeDtypeStruct((NP, CH), _f32),
    )(P, zp, dis, b, Wn)


def _tc_final(P, z5, dis, b5, lw, lb):
    def body(p_r, z_r, dis_r, b_r, lw_r, lb_r, out_r):
        d = dis_r[...]
        h = jnp.maximum(d * (p_r[0] + p_r[1] + z_r[...]) + b_r[...], 0.0)
        out_r[...] = jnp.sum(h * lw_r[...], axis=1, keepdims=True) + lb_r[...]

    return pl.pallas_call(
        body, grid=(_GRID,),
        in_specs=[pl.BlockSpec((2, _BN, CH), lambda i: (0, i, 0)),
                  _rows(CH), _rows(1), _full((1, CH)), _full((1, CH)),
                  _full((1, 1))],
        out_specs=pl.BlockSpec((_BN, 1), lambda i: (i, 0)),
        out_shape=jax.ShapeDtypeStruct((NP, 1), _f32),
    )(P, z5, dis, b5, lw, lb)


def kernel(x, edge_index, batch_size, W1, b1, W2, b2, W3, b3, W4, b4, W5, b5,
           lin_w, lin_b):
    del batch_size
    src = edge_index[0]
    dst = edge_index[1]
    pad = EP - E
    srcp = jnp.concatenate([src, jnp.full((pad,), N, _i32)]).reshape(EROWS, 128)
    dstp = jnp.concatenate([dst, jnp.full((pad,), N, _i32)]).reshape(EROWS, 128)
    src4 = jnp.concatenate([srcp + g * NP for g in range(4)], axis=0)
    zeros = jnp.zeros((RB * 128, CH), _f32)
    ones = jnp.ones((RB * 128, CH), _f32)
    x_pad = jnp.pad(x, ((0, NP - N), (0, CH - 2)))
    W1p = jnp.pad(W1, ((0, CH - 2), (0, 0)))

    PA = _sc_pass(1, False)(ones, dstp, dstp, zeros)
    dis, xs = _tc_a(PA, x_pad)
    PB = _sc_pass(1, True)(xs, srcp, dstp, zeros)
    z2g = _tc_b(PB, xs, dis, W1p, b1.reshape(1, -1), W2)
    zf = z2g.reshape(4 * NP, CH)
    S2a = _sc_pass(2, True)(zf, src4[:2 * EROWS], dstp, zeros)
    S2b = _sc_pass(2, True)(zf, src4[2 * EROWS:], dstp, zeros)
    z3 = _tc_c(S2a.reshape(2, NP, CH), S2b.reshape(2, NP, CH), z2g, dis,
               b2.reshape(1, -1), W3)
    P3 = _sc_pass(1, True)(z3, srcp, dstp, zeros)
    z4 = _tc_mid(P3.reshape(2, NP, CH), z3, dis, b3.reshape(1, -1), W4)
    P4 = _sc_pass(1, True)(z4, srcp, dstp, zeros)
    z5 = _tc_mid(P4.reshape(2, NP, CH), z4, dis, b4.reshape(1, -1), W5)
    P5 = _sc_pass(1, True)(z5, srcp, dstp, zeros)
    y = _tc_final(P5.reshape(2, NP, CH), z5, dis, b5.reshape(1, -1),
                  lin_w.reshape(1, -1), lin_b.reshape(1, 1))
    return y[:N, 0].reshape(50, 1000)


# speedup vs baseline: 10.5973x; 1.0212x over previous
import functools

import jax
import jax.numpy as jnp
from jax import lax
from jax.experimental import pallas as pl
from jax.experimental.pallas import tpu as pltpu
from jax.experimental.pallas import tpu_sc as plsc

N = 50000
NP = 51200
E = 800000
EP = 819200
EROWS = EP // 128
ROWS_T = NP // 16
RB = 4
CH = 32

_f32 = jnp.float32
_i32 = jnp.int32

_BN = 512
_GRID = NP // _BN

@functools.lru_cache(maxsize=None)
def _sc_pass(num_groups, do_gather):
    mesh = plsc.VectorSubcoreMesh(core_axis_name="c", subcore_axis_name="s")
    out_type = jax.ShapeDtypeStruct((2 * NP, CH), _f32)
    scratch = [
        pltpu.VMEM_SHARED((NP, CH), _f32),
        pltpu.VMEM((RB, 128), _i32),
        pltpu.VMEM((RB, 128), _i32),
        pltpu.VMEM((RB * 128, CH), _f32),
        pltpu.SemaphoreType.DMA,
        pltpu.SemaphoreType.DMA,
    ]

    @functools.partial(
        pl.kernel, out_type=out_type, mesh=mesh, scratch_types=scratch,
        compiler_params=pltpu.CompilerParams(use_tc_tiling_on_sc=False))
    def k(z_hbm, src_hbm, dst_hbm, zeros_hbm, out_hbm,
          acc, src_v, dst_v, gbuf, sem_g, sem_s):
        c = lax.axis_index("c")
        s = lax.axis_index("s")
        pltpu.sync_copy(zeros_hbm, gbuf)
        for kk in range(ROWS_T // 400):
            pltpu.sync_copy(gbuf.at[pl.ds(0, 400)],
                            acc.at[pl.ds(s * ROWS_T + kk * 400, 400)])
        plsc.subcore_barrier()
        if not do_gather:
            pltpu.sync_copy(z_hbm, gbuf)
        if num_groups == 1:
            w = s * 2 + c
            row0 = w * (EROWS // 32)
            drow0 = row0
            iters = EROWS // 32 // RB
        else:
            drow0 = s * (EROWS // 16)
            row0 = c * EROWS + drow0
            iters = EROWS // 16 // RB
        obase = c * NP

        def body(i, carry):
            r0 = row0 + i * RB
            rd = drow0 + i * RB
            pltpu.sync_copy(src_hbm.at[pl.ds(r0, RB)], src_v)
            pltpu.sync_copy(dst_hbm.at[pl.ds(rd, RB)], dst_v)
            if do_gather:
                hs = [pltpu.async_copy(z_hbm.at[src_v.at[j]],
                                       gbuf.at[pl.ds(j * 128, 128)],
                                       sem_g)
                      for j in range(RB)]
                for h in hs:
                    h.wait()
            ss = [pltpu.async_copy(gbuf.at[pl.ds(j * 128, 128)],
                                   acc.at[dst_v.at[j]], sem_s, add=True)
                  for j in range(RB)]
            for h in ss:
                h.wait()
            return carry

        lax.fori_loop(0, iters, body, 0)
        plsc.subcore_barrier()
        for kk in range(ROWS_T // 400):
            r = s * ROWS_T + kk * 400
            pltpu.sync_copy(acc.at[pl.ds(r, 400)], gbuf.at[pl.ds(0, 400)])
            pltpu.sync_copy(gbuf.at[pl.ds(0, 400)],
                            out_hbm.at[pl.ds(obase + r, 400)])

    return k


def _rows(w):
    return pl.BlockSpec((_BN, w), lambda i: (i, 0))


def _full(shape):
    return pl.BlockSpec(shape, lambda i: (0,) * len(shape))


def _tc_a(PA, x_pad):
    def body(p0, p1, xp, dis_o, xs_o):
        i = pl.program_id(0)
        indeg = p0[:, 0:1] + p1[:, 0:1]
        rows = i * _BN + lax.broadcasted_iota(_i32, (_BN, 1), 0)
        dis = jnp.where(rows < N, lax.rsqrt(indeg + 1.0), 0.0)
        dis_o[...] = dis
        xs_o[...] = xp[...] * dis

    return pl.pallas_call(
        body, grid=(_GRID,),
        in_specs=[pl.BlockSpec((_BN, CH), lambda i: (i, 0)),
                  pl.BlockSpec((_BN, CH), lambda i: (i + _GRID, 0)),
                  _rows(CH)],
        out_specs=[pl.BlockSpec((_BN, 1), lambda i: (i, 0)), _rows(CH)],
        out_shape=[jax.ShapeDtypeStruct((NP, 1), _f32),
                   jax.ShapeDtypeStruct((NP, CH), _f32)],
    )(PA, PA, x_pad)


def _tc_b(PB, xs, dis, W1p, b1, W2):
    def body(p0, p1, xs_r, dis_r, w1_r, b1_r, w2_r, out_r):
        d = dis_r[...]
        agg = d * (p0[...] + p1[...] + xs_r[...])
        h1 = jnp.maximum(
            jnp.dot(agg, w1_r[...], preferred_element_type=_f32) + b1_r[...],
            0.0)
        z2 = jnp.dot(h1, w2_r[...], preferred_element_type=_f32) * d
        for g in range(4):
            out_r[g] = z2[:, g * CH:(g + 1) * CH]

    return pl.pallas_call(
        body, grid=(_GRID,),
        in_specs=[pl.BlockSpec((_BN, CH), lambda i: (i, 0)),
                  pl.BlockSpec((_BN, CH), lambda i: (i + _GRID, 0)),
                  _rows(CH), _rows(1),
                  _full((CH, 128)), _full((1, 128)), _full((128, 128))],
        out_specs=pl.BlockSpec((4, _BN, CH), lambda i: (0, i, 0)),
        out_shape=jax.ShapeDtypeStruct((4, NP, CH), _f32),
    )(PB, PB, xs, dis, W1p, b1, W2)


def _tc_c(S2a, S2b, z2g, dis, b2, W3):
    def body(sa_r, sb_r, z2_r, dis_r, b2_r, w3_r, out_r):
        d = dis_r[...]
        s2 = jnp.concatenate([sa_r[0], sa_r[1], sb_r[0], sb_r[1]], axis=1)
        z2 = jnp.concatenate([z2_r[g] for g in range(4)], axis=1)
        h2 = jnp.maximum(d * (s2 + z2) + b2_r[...], 0.0)
        out_r[...] = jnp.dot(h2, w3_r[...], preferred_element_type=_f32) * d

    return pl.pallas_call(
        body, grid=(_GRID,),
        in_specs=[pl.BlockSpec((2, _BN, CH), lambda i: (0, i, 0)),
                  pl.BlockSpec((2, _BN, CH), lambda i: (0, i, 0)),
                  pl.BlockSpec((4, _BN, CH), lambda i: (0, i, 0)),
                  _rows(1), _full((1, 128)), _full((128, CH))],
        out_specs=_rows(CH),
        out_shape=jax.ShapeDtypeStruct((NP, CH), _f32),
    )(S2a, S2b, z2g, dis, b2, W3)


def _tc_mid(P, zp, dis, b, Wn):
    def body(p_r, z_r, dis_r, b_r, w_r, out_r):
        d = dis_r[...]
        h = jnp.maximum(d * (p_r[0] + p_r[1] + z_r[...]) + b_r[...], 0.0)
        out_r[...] = jnp.dot(h, w_r[...], preferred_element_type=_f32) * d

    return pl.pallas_call(
        body, grid=(_GRID,),
        in_specs=[pl.BlockSpec((2, _BN, CH), lambda i: (0, i, 0)),
                  _rows(CH), _rows(1), _full((1, CH)), _full((CH, CH))],
        out_specs=_rows(CH),
        out_shape=jax.ShapeDtypeStruct((NP, CH), _f32),
    )(P, zp, dis, b, Wn)


def _tc_final(P, z5, dis, b5, lw, lb):
    def body(p_r, z_r, dis_r, b_r, lw_r, lb_r, out_r):
        d = dis_r[...]
        h = jnp.maximum(d * (p_r[0] + p_r[1] + z_r[...]) + b_r[...], 0.0)
        out_r[...] = jnp.sum(h * lw_r[...], axis=1, keepdims=True) + lb_r[...]

    return pl.pallas_call(
        body, grid=(_GRID,),
        in_specs=[pl.BlockSpec((2, _BN, CH), lambda i: (0, i, 0)),
                  _rows(CH), _rows(1), _full((1, CH)), _full((1, CH)),
                  _full((1, 1))],
        out_specs=pl.BlockSpec((_BN, 1), lambda i: (i, 0)),
        out_shape=jax.ShapeDtypeStruct((NP, 1), _f32),
    )(P, z5, dis, b5, lw, lb)


def kernel(x, edge_index, batch_size, W1, b1, W2, b2, W3, b3, W4, b4, W5, b5,
           lin_w, lin_b):
    del batch_size
    src = edge_index[0]
    dst = edge_index[1]
    pad = EP - E
    srcp = jnp.concatenate([src, jnp.full((pad,), N, _i32)]).reshape(EROWS, 128)
    dstp = jnp.concatenate([dst, jnp.full((pad,), N, _i32)]).reshape(EROWS, 128)
    src4 = jnp.concatenate([srcp + g * NP for g in range(4)], axis=0)
    zeros = jnp.zeros((RB * 128, CH), _f32)
    ones = jnp.ones((RB * 128, CH), _f32)
    x_pad = jnp.pad(x, ((0, NP - N), (0, CH - 2)))
    W1p = jnp.pad(W1, ((0, CH - 2), (0, 0)))

    PA = _sc_pass(1, False)(ones, dstp, dstp, zeros)
    dis, xs = _tc_a(PA, x_pad)
    PB = _sc_pass(1, True)(xs, srcp, dstp, zeros)
    z2g = _tc_b(PB, xs, dis, W1p, b1.reshape(1, -1), W2)
    zf = z2g.reshape(4 * NP, CH)
    S2a = _sc_pass(2, True)(zf, src4[:2 * EROWS], dstp, zeros)
    S2b = _sc_pass(2, True)(zf, src4[2 * EROWS:], dstp, zeros)
    z3 = _tc_c(S2a.reshape(2, NP, CH), S2b.reshape(2, NP, CH), z2g, dis,
               b2.reshape(1, -1), W3)
    P3 = _sc_pass(1, True)(z3, srcp, dstp, zeros)
    z4 = _tc_mid(P3.reshape(2, NP, CH), z3, dis, b3.reshape(1, -1), W4)
    P4 = _sc_pass(1, True)(z4, srcp, dstp, zeros)
    z5 = _tc_mid(P4.reshape(2, NP, CH), z4, dis, b4.reshape(1, -1), W5)
    P5 = _sc_pass(1, True)(z5, srcp, dstp, zeros)
    y = _tc_final(P5.reshape(2, NP, CH), z5, dis, b5.reshape(1, -1),
                  lin_w.reshape(1, -1), lin_b.reshape(1, 1))
    return y[:N, 0].reshape(50, 1000)
